# Initial kernel scaffold; baseline (speedup 1.0000x reference)
import jax
import jax.numpy as jnp
from jax.experimental import pallas as pl


def kernel(x, edge_index, batch, w1, b1, w2, b2, w3, b3, t0c1w, t0c1b, t0c2w, t0c2b, t0dw, t0db, t1c1w, t1c1b, t1c2w, t1c2b, t2c1w, t2c1b, t2c2w, t2c2b, t3c1w, t3c1b, t3c2w, t3c2b, f1w, f1b, f2w, f2b, f3w, f3b):
    raise NotImplementedError("write your pallas kernel here")


# speedup vs baseline: 1.3082x; 1.3082x over previous
import functools

import jax
import jax.numpy as jnp
from jax import lax
from jax.experimental import pallas as pl
from jax.experimental.pallas import tpu as pltpu
from jax.experimental.pallas import tpu_sc as plsc

N = 10000
E = 320000
DIN = 128
PRE = 256
G = 48
TCN = 64
KSZ = 3

NC, NS, L = 2, 16, 16
NW = NC * NS
FPP = 4
NPASS = PRE // (NW * FPP)
NP = 10240
EP = NW * 10240
ECH = 512
NCH = EP // ECH
EPO = 4
DCH = 512
EPT = EP // NW

_SC_CACHE = {}


def _sc_kernels():
    if _SC_CACHE:
        return _SC_CACHE
    mesh = plsc.VectorSubcoreMesh(core_axis_name="c", subcore_axis_name="s",
                                  num_cores=NC, num_subcores=NS)

    @functools.partial(
        pl.kernel, mesh=mesh,
        compiler_params=pltpu.CompilerParams(needs_layout_passes=False),
        out_type=jax.ShapeDtypeStruct((NW * NP, ), jnp.float32),
        scratch_types=[
            pltpu.VMEM((DCH,), jnp.int32),
            pltpu.VMEM((NP,), jnp.float32),
        ],
    )
    def _sc_degree(col_hbm, deg_out, colbuf, hist):
        cid = lax.axis_index("c")
        sid = lax.axis_index("s")
        t = cid * NS + sid
        zero = jnp.zeros((L,), jnp.float32)
        one = jnp.full((L,), 1.0, jnp.float32)

        def zbody(i, carry):
            hist[pl.ds(i * L, L)] = zero
            return carry

        lax.fori_loop(0, NP // L, zbody, 0)

        def cbody(g, carry):
            pltpu.sync_copy(col_hbm.at[pl.ds(t * EPT + g * DCH, DCH)], colbuf)
            for i in range(DCH // L):
                cv = colbuf[pl.ds(i * L, L)]
                plsc.addupdate_scatter(hist, [cv], one)
            return carry

        lax.fori_loop(0, EPT // DCH, cbody, 0)
        pltpu.sync_copy(hist, deg_out.at[pl.ds(t * NP, NP)])

    @functools.partial(
        pl.kernel, mesh=mesh,
        compiler_params=pltpu.CompilerParams(needs_layout_passes=False),
        out_type=jax.ShapeDtypeStruct((NW * NPASS * FPP * NP,), jnp.float32),
        scratch_types=[
            pltpu.VMEM((ECH,), jnp.int32),
            pltpu.VMEM((ECH,), jnp.int32),
            pltpu.VMEM((FPP * NPASS, N), jnp.float32),
            pltpu.VMEM((FPP * NP,), jnp.float32),
        ],
    )
    def _sc_edge(yt_hbm, row_hbm, col_hbm, zt_out, rowbuf, colbuf, ytl, acc):
        cid = lax.axis_index("c")
        sid = lax.axis_index("s")
        t = cid * NS + sid
        zero = jnp.zeros((L,), jnp.float32)
        lane = lax.iota(jnp.int32, L)
        iq = lane // FPP
        fq = lane % FPP

        nrows = FPP * NPASS
        pltpu.sync_copy(yt_hbm.at[pl.ds(t * nrows, nrows)], ytl)

        fqNP = fq * NP
        for p in range(NPASS):
            def zbody(i, carry):
                acc[pl.ds(i * L, L)] = zero
                return carry

            lax.fori_loop(0, FPP * NP // L, zbody, 0)
            frow = fq + p * FPP

            def cbody(g, carry):
                pltpu.sync_copy(row_hbm.at[pl.ds(g * ECH, ECH)], rowbuf)
                pltpu.sync_copy(col_hbm.at[pl.ds(g * ECH, ECH)], colbuf)

                def ebody(i, carry2):
                    sel = iq + i * EPO
                    pr = plsc.load_gather(rowbuf, [sel])
                    pc = plsc.load_gather(colbuf, [sel])
                    vals = plsc.load_gather(ytl, [frow, pr])
                    plsc.addupdate_scatter(acc, [fqNP + pc], vals)
                    return carry2

                lax.fori_loop(0, ECH // EPO, ebody, 0)
                return carry

            lax.fori_loop(0, NCH, cbody, 0)
            pltpu.sync_copy(
                acc, zt_out.at[pl.ds((t * NPASS + p) * FPP * NP, FPP * NP)])

    _SC_CACHE.update(deg=_sc_degree, edge=_sc_edge)
    return _SC_CACHE


MB = 1000


def _tc_dis_body(d_ref, dis_ref):
    deg = 1.0 + jnp.sum(d_ref[...], axis=1, keepdims=True)
    dis_ref[...] = jnp.broadcast_to(lax.rsqrt(deg), (MB, PRE))


def _tc_dis(dmat):
    return pl.pallas_call(
        _tc_dis_body,
        grid=(N // MB,),
        in_specs=[pl.BlockSpec((MB, NW), lambda i: (i, 0))],
        out_specs=pl.BlockSpec((MB, PRE), lambda i: (i, 0)),
        out_shape=jax.ShapeDtypeStruct((N, PRE), jnp.float32),
    )(dmat)


def _tc_mm_body(h_ref, w_ref, dis_ref, yt_ref):
    y = jnp.dot(h_ref[...], w_ref[...], preferred_element_type=jnp.float32)
    yt_ref[...] = (y * dis_ref[...]).T


def _tc_mm(h, w, dis):
    return pl.pallas_call(
        _tc_mm_body,
        out_shape=jax.ShapeDtypeStruct((PRE, N), jnp.float32),
    )(h, w, dis)


def _tc_post_body(zt_ref, yt_ref, dis_ref, b_ref, h_ref):
    h_ref[...] = jnp.maximum(
        (zt_ref[...].T + yt_ref[...].T) * dis_ref[...] + b_ref[...], 0.0)


def _tc_post(zt, yt, dis, b):
    return pl.pallas_call(
        _tc_post_body,
        out_shape=jax.ShapeDtypeStruct((N, PRE), jnp.float32),
    )(zt, yt, dis, b)


def _tc_pool_body(h_ref, b_ref, sum_ref, max_ref, cnt_ref):
    i = pl.program_id(0)
    h = h_ref[...]
    bb = b_ref[...][:, 0:1]
    sums, maxs, cnts = [], [], []
    for g in range(G):
        m = bb == g
        mf = m.astype(jnp.float32)
        sums.append(jnp.sum(h * mf, axis=0, keepdims=True))
        maxs.append(jnp.max(jnp.where(m, h, 0.0), axis=0, keepdims=True))
        cnts.append(jnp.sum(mf, axis=0, keepdims=True))
    ps = jnp.concatenate(sums, axis=0)
    pm = jnp.concatenate(maxs, axis=0)
    pc = jnp.broadcast_to(jnp.concatenate(cnts, axis=0), (G, PRE))

    @pl.when(i == 0)
    def _():
        sum_ref[...] = ps
        max_ref[...] = pm
        cnt_ref[...] = pc

    @pl.when(i > 0)
    def _():
        sum_ref[...] = sum_ref[...] + ps
        max_ref[...] = jnp.maximum(max_ref[...], pm)
        cnt_ref[...] = cnt_ref[...] + pc


def _tc_pool(h, batchb):
    return pl.pallas_call(
        _tc_pool_body,
        grid=(N // MB,),
        in_specs=[
            pl.BlockSpec((MB, PRE), lambda i: (i, 0)),
            pl.BlockSpec((MB, 128), lambda i: (i, 0)),
        ],
        out_specs=[
            pl.BlockSpec((G, PRE), lambda i: (0, 0)),
            pl.BlockSpec((G, PRE), lambda i: (0, 0)),
            pl.BlockSpec((G, PRE), lambda i: (0, 0)),
        ],
        out_shape=[
            jax.ShapeDtypeStruct((G, PRE), jnp.float32),
            jax.ShapeDtypeStruct((G, PRE), jnp.float32),
            jax.ShapeDtypeStruct((G, PRE), jnp.float32),
        ],
    )(h, batchb)


def _shift(x, s):
    if s == 0:
        return x
    cin, t = x.shape
    return jnp.concatenate(
        [jnp.zeros((cin, s), jnp.float32), x[:, : t - s]], axis=1)


def _head_body(sums_ref, maxs_ref, cnts_ref, *refs):
    (t0c1k0, t0c1k1, t0c1k2, t0c1b, t0c2k0, t0c2k1, t0c2k2, t0c2b,
     t0d, t0db,
     t1c1k0, t1c1k1, t1c1k2, t1c1b, t1c2k0, t1c2k1, t1c2k2, t1c2b,
     t2c1k0, t2c1k1, t2c1k2, t2c1b, t2c2k0, t2c2k1, t2c2k2, t2c2b,
     t3c1k0, t3c1k1, t3c1k2, t3c1b, t3c2k0, t3c2k1, t3c2k2, t3c2b,
     f1w, f1b, f2w, f2b, f3w, f3b, out_ref) = refs

    s = sums_ref[...]
    mx = maxs_ref[...]
    cnt = cnts_ref[...][:, 0:1]
    mean = s / jnp.maximum(cnt, 1.0)
    g = jnp.concatenate([mean, mx, s], axis=1)

    def conv(x, taps, b_ref, dil):
        acc = None
        for k, tap in enumerate(taps):
            xs = _shift(x, (KSZ - 1 - k) * dil)
            c = jnp.dot(tap[...], xs, preferred_element_type=jnp.float32)
            acc = c if acc is None else acc + c
        return acc + b_ref[...].reshape(TCN, 1)

    def block(t, c1, c1b, c2, c2b, d, db, dil):
        o = jnp.maximum(conv(t, c1, c1b, dil), 0.0)
        o = jnp.maximum(conv(o, c2, c2b, dil), 0.0)
        if d is None:
            res = t
        else:
            res = jnp.dot(d[...], t, preferred_element_type=jnp.float32) \
                + db[...].reshape(TCN, 1)
        return jnp.maximum(o + res, 0.0)

    t = block(g, (t0c1k0, t0c1k1, t0c1k2), t0c1b,
              (t0c2k0, t0c2k1, t0c2k2), t0c2b, t0d, t0db, 1)
    t = block(t, (t1c1k0, t1c1k1, t1c1k2), t1c1b,
              (t1c2k0, t1c2k1, t1c2k2), t1c2b, None, None, 2)
    t = block(t, (t2c1k0, t2c1k1, t2c1k2), t2c1b,
              (t2c2k0, t2c2k1, t2c2k2), t2c2b, None, None, 4)
    t = block(t, (t3c1k0, t3c1k1, t3c1k2), t3c1b,
              (t3c2k0, t3c2k1, t3c2k2), t3c2b, None, None, 8)

    o = jnp.maximum(jnp.dot(t, f1w[...], preferred_element_type=jnp.float32)
                    + f1b[...].reshape(1, -1), 0.0)
    o = jnp.maximum(jnp.dot(o, f2w[...], preferred_element_type=jnp.float32)
                    + f2b[...].reshape(1, -1), 0.0)
    o = jnp.dot(o, f3w[...], preferred_element_type=jnp.float32) \
        + f3b[...].reshape(1, -1)
    out_ref[...] = o


def _tc_head(sums, maxs, cnts, wlist):
    return pl.pallas_call(
        _head_body,
        out_shape=jax.ShapeDtypeStruct((TCN, 1), jnp.float32),
    )(sums, maxs, cnts, *wlist)


def kernel(x, edge_index, batch, w1, b1, w2, b2, w3, b3,
           t0c1w, t0c1b, t0c2w, t0c2b, t0dw, t0db,
           t1c1w, t1c1b, t1c2w, t1c2b,
           t2c1w, t2c1b, t2c2w, t2c2b,
           t3c1w, t3c1b, t3c2w, t3c2b,
           f1w, f1b, f2w, f2b, f3w, f3b):
    row = edge_index[0].astype(jnp.int32)
    col = edge_index[1].astype(jnp.int32)
    batch = batch.astype(jnp.int32)
    npad = EP - E
    rowp = jnp.concatenate([row, jnp.zeros((npad,), jnp.int32)])
    colp = jnp.concatenate([col, jnp.full((npad,), NP - 1, jnp.int32)])

    sck = _sc_kernels()
    degf = sck["deg"](colp)
    dmat = degf.reshape(NW, NP)[:, :N].T
    dis = _tc_dis(dmat)

    h = x
    for w, b in ((w1, b1), (w2, b2), (w3, b3)):
        yt = _tc_mm(h, w, dis)
        ztf = sck["edge"](yt, rowp, colp)
        zt = ztf.reshape(PRE, NP)[:, :N]
        h = _tc_post(zt, yt, dis, b.reshape(1, PRE))

    batchb = jnp.broadcast_to(batch[:, None], (N, 128))
    sums, maxs, cnts = _tc_pool(h, batchb)

    wlist = []
    for cw, cb in ((t0c1w, t0c1b), (t0c2w, t0c2b)):
        wlist += [cw[:, :, 0], cw[:, :, 1], cw[:, :, 2], cb]
    wlist += [t0dw[:, :, 0], t0db]
    for cw, cb in ((t1c1w, t1c1b), (t1c2w, t1c2b),
                   (t2c1w, t2c1b), (t2c2w, t2c2b),
                   (t3c1w, t3c1b), (t3c2w, t3c2b)):
        wlist += [cw[:, :, 0], cw[:, :, 1], cw[:, :, 2], cb]
    wlist += [f1w, f1b, f2w, f2b, f3w, f3b]

    o = _tc_head(sums, maxs, cnts, wlist)
    return o[:, 0]
